# baseline (speedup 1.0000x reference)
import functools

import jax
import jax.numpy as jnp
from jax import lax
from jax.experimental import pallas as pl
from jax.experimental.pallas import tpu as pltpu
from jax.experimental.pallas import tpu_sc as plsc

_B, _Q, _C = 32, 900, 80
_L = 16
_CH = 16
_NFULL = _Q // _CH
_NCH = _NFULL + 1
_QP = _NCH * _CH
_NG = _C // _L
_NEG = -3.0e38


def _insert4(v, idx, c1, c2, c3, c4, j1, j2, j3, j4):
    g = v > c1
    nc1 = jnp.where(g, v, c1)
    nj1 = jnp.where(g, idx, j1)
    v, idx = jnp.where(g, c1, v), jnp.where(g, j1, idx)
    g = v > c2
    nc2 = jnp.where(g, v, c2)
    nj2 = jnp.where(g, idx, j2)
    v, idx = jnp.where(g, c2, v), jnp.where(g, j2, idx)
    g = v > c3
    nc3 = jnp.where(g, v, c3)
    nj3 = jnp.where(g, idx, j3)
    v, idx = jnp.where(g, c3, v), jnp.where(g, j3, idx)
    g = v > c4
    nc4 = jnp.where(g, v, c4)
    nj4 = jnp.where(g, idx, j4)
    return nc1, nc2, nc3, nc4, nj1, nj2, nj3, nj4


def _sc_body(fp_hbm, bb_hbm, sums_hbm, pair_hbm, fp_v, bb_v, sums_v, pair_v):
    b = lax.axis_index("s") * 2 + lax.axis_index("c")
    pltpu.sync_copy(fp_hbm.at[pl.ds(b * (_Q * _C), _Q * _C)],
                    fp_v.at[pl.ds(0, _Q * _C)])
    pltpu.sync_copy(bb_hbm.at[pl.ds(b * (_Q * 4), _Q * 4)], bb_v)

    neg = jnp.full((_L,), _NEG, jnp.float32)
    zero = jnp.zeros((_L,), jnp.float32)
    zi = jnp.zeros((_L,), jnp.int32)
    lane = lax.iota(jnp.int32, _L)

    def _pad(i, carry):
        fp_v[pl.ds(_Q * _C + i * _L, _L)] = neg
        return carry

    lax.fori_loop(0, (_QP - _Q) * _C // _L, _pad, 0)

    for g in range(_NG):
        col0 = g * _L

        def chunk_body(j, carry, col0=col0):
            acc, c1, c2, c3, c4, j1, j2, j3, j4 = carry
            m = neg
            base = j * (_CH * _C) + col0
            for t in range(_CH):
                v = fp_v[pl.ds(base + t * _C, _L)]
                acc = acc + v
                m = jnp.maximum(m, v)
            c1, c2, c3, c4, j1, j2, j3, j4 = _insert4(
                m, zi + j, c1, c2, c3, c4, j1, j2, j3, j4)
            return (acc, c1, c2, c3, c4, j1, j2, j3, j4)

        carry = (zero, neg, neg, neg, neg, zi, zi, zi, zi)
        acc, c1, c2, c3, c4, j1, j2, j3, j4 = lax.fori_loop(
            0, _NFULL, chunk_body, carry)

        m = neg
        base = _NFULL * _CH * _C + col0
        for t in range(_Q - _NFULL * _CH):
            v = fp_v[pl.ds(base + t * _C, _L)]
            acc = acc + v
            m = jnp.maximum(m, v)
        c1, c2, c3, c4, j1, j2, j3, j4 = _insert4(
            m, zi + _NFULL, c1, c2, c3, c4, j1, j2, j3, j4)
        sums_v[pl.ds(col0, _L)] = acc

        sa, sb, sc, sd = j1, j2, j3, j4
        sa, sb = jnp.minimum(sa, sb), jnp.maximum(sa, sb)
        sc, sd = jnp.minimum(sc, sd), jnp.maximum(sc, sd)
        sa, sc = jnp.minimum(sa, sc), jnp.maximum(sa, sc)
        sb, sd = jnp.minimum(sb, sd), jnp.maximum(sb, sd)
        sb, sc = jnp.minimum(sb, sc), jnp.maximum(sb, sc)

        colv = lane + col0
        carry2 = (neg, neg, neg, neg, zi, zi, zi, zi)
        for jk in (sa, sb, sc, sd):
            rowbase = jk * _CH

            def resc(t, carry, rowbase=rowbase, colv=colv):
                m1, m2, m3, m4, i1, i2, i3, i4 = carry
                rows = rowbase + t
                v = plsc.load_gather(fp_v, [rows * _C + colv])
                return _insert4(v, rows, m1, m2, m3, m4, i1, i2, i3, i4)

            carry2 = lax.fori_loop(0, _CH, resc, carry2)
        m1, m2, m3, m4, i1, i2, i3, i4 = carry2

        g0 = [plsc.load_gather(bb_v, [i1 * 4 + d]) for d in range(4)]
        s = zero
        for ik in (i2, i3, i4):
            for d in range(4):
                s = s + jnp.abs(plsc.load_gather(bb_v, [ik * 4 + d]) - g0[d])
        pair_v[pl.ds(col0, _L)] = s * 0.25

    pltpu.sync_copy(sums_v, sums_hbm.at[pl.ds(b * _C, _C)])
    pltpu.sync_copy(pair_v, pair_hbm.at[pl.ds(b * _C, _C)])


_sc_topk_cache = []


def _get_sc_topk():
    if not _sc_topk_cache:
        mesh = plsc.VectorSubcoreMesh(
            core_axis_name="c", subcore_axis_name="s",
            num_cores=2, num_subcores=16)
        _sc_topk_cache.append(pl.kernel(
            _sc_body,
            out_type=(jax.ShapeDtypeStruct((_B * _C,), jnp.float32),
                      jax.ShapeDtypeStruct((_B * _C,), jnp.float32)),
            mesh=mesh,
            scratch_types=[
                pltpu.VMEM((_QP * _C,), jnp.float32),
                pltpu.VMEM((_Q * 4,), jnp.float32),
                pltpu.VMEM((_C,), jnp.float32),
                pltpu.VMEM((_C,), jnp.float32),
            ],
            compiler_params=pltpu.CompilerParams(
                needs_layout_passes=False,
                use_tc_tiling_on_sc=False,
            ),
        ))
    return _sc_topk_cache[0]


def _finish_body(sums_ref, pair_ref, lab_ref, a1_ref, a2_ref, warm_ref,
                 tot_ref, mil_ref, areg_ref, box_ref):
    s = sums_ref[...]
    labv = lab_ref[...]
    preds = jnp.clip(s, 0.0, 1.0)
    log_p = jnp.maximum(jnp.log(preds), -100.0)
    log_1mp = jnp.maximum(jnp.log(1.0 - preds), -100.0)
    mil = -jnp.mean(labv * log_p + (1.0 - labv) * log_1mp)
    a1 = a1_ref[...]
    a2 = a2_ref[...]
    areg = 0.01 * 0.5 * (jnp.mean((a1 - 0.5) ** 2)
                         + jnp.mean((a2 - 0.5) ** 2))
    warm = warm_ref[0, 0]
    pairsum = jnp.sum(pair_ref[...] * labv)
    valid = jnp.sum(labv) * 3.0
    box = warm * (pairsum / jnp.maximum(valid, 1.0))
    tot_ref[0, 0] = mil + areg + box
    mil_ref[0, 0] = mil
    areg_ref[0, 0] = areg
    box_ref[0, 0] = box


def kernel(final_prob, bboxes, alpha_1, alpha_2, image_labels,
           current_epoch, warmup_epochs):
    fp = final_prob.reshape(_B * _Q * _C)
    bb = bboxes.reshape(_B * _Q * 4)
    sums, pair = _get_sc_topk()(fp, bb)
    sums = sums.reshape(_B, _C)
    pair = pair.reshape(_B, _C)
    labv = image_labels.astype(jnp.float32)
    a1 = alpha_1.reshape(1, _B)
    a2 = alpha_2.reshape(1, _B)
    warm = (jnp.asarray(current_epoch, jnp.int32)
            >= jnp.asarray(warmup_epochs, jnp.int32))
    warm = warm.astype(jnp.float32).reshape(1, 1)
    tot, mil, areg, box = pl.pallas_call(
        _finish_body,
        out_shape=[jax.ShapeDtypeStruct((1, 1), jnp.float32)] * 4,
        out_specs=[pl.BlockSpec(memory_space=pltpu.SMEM)] * 4,
    )(sums, pair, labv, a1, a2, warm)
    return (tot[0, 0], mil[0, 0], areg[0, 0], box[0, 0])


# speedup vs baseline: 1.1359x; 1.1359x over previous
import functools

import jax
import jax.numpy as jnp
from jax import lax
from jax.experimental import pallas as pl
from jax.experimental.pallas import tpu as pltpu
from jax.experimental.pallas import tpu_sc as plsc

_B, _Q, _C = 32, 900, 80
_L = 16
_CH = 16
_NCH = 57
_QP = _NCH * _CH
_NT = _QP // 8
_TW = 1024
_BW = _NT * _TW
_NG = _C // _L
_NEG = -3.0e38


def _retile_body(x_ref, fp4_ref, sums_ref):
    x = x_ref[0]
    sums_ref[0, 0] = jnp.sum(x, axis=0)
    head = x[0:896].reshape(112, 8, 80)
    fp4_ref[0, 0:112, :, 0:80] = head
    tail = jnp.concatenate(
        [x[896:900], jnp.full((_QP - _Q, _C), _NEG, jnp.float32)], axis=0)
    fp4_ref[0, 112:114, :, 0:80] = tail.reshape(2, 8, 80)


def _insert4(v, idx, c1, c2, c3, c4, j1, j2, j3, j4):
    g = v > c1
    nc1 = jnp.where(g, v, c1)
    nj1 = jnp.where(g, idx, j1)
    v, idx = jnp.where(g, c1, v), jnp.where(g, j1, idx)
    g = v > c2
    nc2 = jnp.where(g, v, c2)
    nj2 = jnp.where(g, idx, j2)
    v, idx = jnp.where(g, c2, v), jnp.where(g, j2, idx)
    g = v > c3
    nc3 = jnp.where(g, v, c3)
    nj3 = jnp.where(g, idx, j3)
    v, idx = jnp.where(g, c3, v), jnp.where(g, j3, idx)
    g = v > c4
    nc4 = jnp.where(g, v, c4)
    nj4 = jnp.where(g, idx, j4)
    return nc1, nc2, nc3, nc4, nj1, nj2, nj3, nj4


def _sc_body(fp_hbm, bb_hbm, pair_hbm, fp_v, bb_v, pair_v):
    b = lax.axis_index("s") * 2 + lax.axis_index("c")
    pltpu.sync_copy(fp_hbm.at[pl.ds(b * _BW, _BW)], fp_v)
    pltpu.sync_copy(bb_hbm.at[pl.ds(b * (_Q * 4), _Q * 4)], bb_v)

    neg = jnp.full((_L,), _NEG, jnp.float32)
    zero = jnp.zeros((_L,), jnp.float32)
    zi = jnp.zeros((_L,), jnp.int32)
    lane = lax.iota(jnp.int32, _L)

    for g in range(_NG):
        col0 = g * _L

        def chunk_body(j, carry, col0=col0):
            c1, c2, c3, c4, j1, j2, j3, j4 = carry
            m = neg
            base = j * (2 * _TW) + col0
            for t in range(_CH):
                off = base + (t // 8) * _TW + (t % 8) * 128
                m = jnp.maximum(m, fp_v[pl.ds(off, _L)])
            return _insert4(m, zi + j, c1, c2, c3, c4, j1, j2, j3, j4)

        carry = (neg, neg, neg, neg, zi, zi, zi, zi)
        c1, c2, c3, c4, j1, j2, j3, j4 = lax.fori_loop(
            0, _NCH, chunk_body, carry)

        sa, sb, sc, sd = j1, j2, j3, j4
        sa, sb = jnp.minimum(sa, sb), jnp.maximum(sa, sb)
        sc, sd = jnp.minimum(sc, sd), jnp.maximum(sc, sd)
        sa, sc = jnp.minimum(sa, sc), jnp.maximum(sa, sc)
        sb, sd = jnp.minimum(sb, sd), jnp.maximum(sb, sd)
        sb, sc = jnp.minimum(sb, sc), jnp.maximum(sb, sc)

        colv = lane + col0
        carry2 = (neg, neg, neg, neg, zi, zi, zi, zi)
        for jk in (sa, sb, sc, sd):
            rowbase = jk * _CH
            addrbase = jk * (2 * _TW) + colv

            def resc(t, carry, rowbase=rowbase, addrbase=addrbase):
                m1, m2, m3, m4, i1, i2, i3, i4 = carry
                off = (t // 8) * _TW + (t % 8) * 128
                v = plsc.load_gather(fp_v, [addrbase + off])
                return _insert4(v, rowbase + t,
                                m1, m2, m3, m4, i1, i2, i3, i4)

            carry2 = lax.fori_loop(0, _CH, resc, carry2)
        m1, m2, m3, m4, i1, i2, i3, i4 = carry2

        g0 = [plsc.load_gather(bb_v, [i1 * 4 + d]) for d in range(4)]
        s = zero
        for ik in (i2, i3, i4):
            for d in range(4):
                s = s + jnp.abs(plsc.load_gather(bb_v, [ik * 4 + d]) - g0[d])
        pair_v[pl.ds(col0, _L)] = s * 0.25

    pltpu.sync_copy(pair_v, pair_hbm.at[pl.ds(b * _C, _C)])


_sc_topk_cache = []


def _get_sc_topk():
    if not _sc_topk_cache:
        mesh = plsc.VectorSubcoreMesh(
            core_axis_name="c", subcore_axis_name="s",
            num_cores=2, num_subcores=16)
        _sc_topk_cache.append(pl.kernel(
            _sc_body,
            out_type=jax.ShapeDtypeStruct((_B * _C,), jnp.float32),
            mesh=mesh,
            scratch_types=[
                pltpu.VMEM((_B * _BW // 32,), jnp.float32),
                pltpu.VMEM((_Q * 4,), jnp.float32),
                pltpu.VMEM((_C,), jnp.float32),
            ],
            compiler_params=pltpu.CompilerParams(
                needs_layout_passes=False,
                use_tc_tiling_on_sc=False,
            ),
        ))
    return _sc_topk_cache[0]


def _finish_body(sums_ref, pair_ref, lab_ref, a1_ref, a2_ref, warm_ref,
                 tot_ref, mil_ref, areg_ref, box_ref):
    s = sums_ref[...]
    labv = lab_ref[...]
    preds = jnp.clip(s, 0.0, 1.0)
    log_p = jnp.maximum(jnp.log(preds), -100.0)
    log_1mp = jnp.maximum(jnp.log(1.0 - preds), -100.0)
    mil = -jnp.mean(labv * log_p + (1.0 - labv) * log_1mp)
    a1 = a1_ref[...]
    a2 = a2_ref[...]
    areg = 0.01 * 0.5 * (jnp.mean((a1 - 0.5) ** 2)
                         + jnp.mean((a2 - 0.5) ** 2))
    warm = warm_ref[0, 0]
    pairsum = jnp.sum(pair_ref[...] * labv)
    valid = jnp.sum(labv) * 3.0
    box = warm * (pairsum / jnp.maximum(valid, 1.0))
    tot_ref[0, 0] = mil + areg + box
    mil_ref[0, 0] = mil
    areg_ref[0, 0] = areg
    box_ref[0, 0] = box


def kernel(final_prob, bboxes, alpha_1, alpha_2, image_labels,
           current_epoch, warmup_epochs):
    fp4, sums = pl.pallas_call(
        _retile_body,
        grid=(_B,),
        in_specs=[pl.BlockSpec((1, _Q, _C), lambda b: (b, 0, 0))],
        out_specs=[
            pl.BlockSpec((1, _NT, 8, 128), lambda b: (b, 0, 0, 0)),
            pl.BlockSpec((1, 1, _C), lambda b: (b, 0, 0)),
        ],
        out_shape=[
            jax.ShapeDtypeStruct((_B, _NT, 8, 128), jnp.float32),
            jax.ShapeDtypeStruct((_B, 1, _C), jnp.float32),
        ],
    )(final_prob)
    sums = sums.reshape(_B, _C)
    bb = bboxes.reshape(_B * _Q * 4)
    pair = _get_sc_topk()(fp4.reshape(_B * _BW), bb)
    pair = pair.reshape(_B, _C)
    labv = image_labels.astype(jnp.float32)
    a1 = alpha_1.reshape(1, _B)
    a2 = alpha_2.reshape(1, _B)
    warm = (jnp.asarray(current_epoch, jnp.int32)
            >= jnp.asarray(warmup_epochs, jnp.int32))
    warm = warm.astype(jnp.float32).reshape(1, 1)
    tot, mil, areg, box = pl.pallas_call(
        _finish_body,
        out_shape=[jax.ShapeDtypeStruct((1, 1), jnp.float32)] * 4,
        out_specs=[pl.BlockSpec(memory_space=pltpu.SMEM)] * 4,
    )(sums, pair, labv, a1, a2, warm)
    return (tot[0, 0], mil[0, 0], areg[0, 0], box[0, 0])


# speedup vs baseline: 1.2628x; 1.1116x over previous
import jax
import jax.numpy as jnp
from jax import lax
from jax.experimental import pallas as pl
from jax.experimental.pallas import tpu as pltpu
from jax.experimental.pallas import tpu_sc as plsc

_B, _Q, _C = 32, 900, 80
_L = 16
_CH = 16
_NCH = 57
_NCT = _C // 8
_NQT = 8
_FPW = _NCT * _NQT * 1024
_BBW = _NQT * 1024
_NG = _C // _L
_NEG = -3.0e38


def _prep_body(x_ref, y_ref, fp5_ref, bb4_ref, sums_ref):
    x = x_ref[0]
    sums_ref[0, 0] = jnp.sum(x, axis=1)
    negpad = jnp.full((8, 128 - (_Q - 896)), _NEG, jnp.float32)
    for ct in range(_NCT):
        xs = x[8 * ct:8 * ct + 8]
        for qt in range(7):
            fp5_ref[0, ct, qt] = xs[:, 128 * qt:128 * qt + 128]
        fp5_ref[0, ct, 7] = jnp.concatenate([xs[:, 896:_Q], negpad], axis=1)
    y = y_ref[0]
    for qt in range(7):
        bb4_ref[0, qt, 0:4, :] = y[:, 128 * qt:128 * qt + 128]
    bb4_ref[0, 7, 0:4, 0:4] = y[:, 896:_Q]


def _insert4(v, idx, c1, c2, c3, c4, j1, j2, j3, j4):
    g = v > c1
    nc1 = jnp.where(g, v, c1)
    nj1 = jnp.where(g, idx, j1)
    v, idx = jnp.where(g, c1, v), jnp.where(g, j1, idx)
    g = v > c2
    nc2 = jnp.where(g, v, c2)
    nj2 = jnp.where(g, idx, j2)
    v, idx = jnp.where(g, c2, v), jnp.where(g, j2, idx)
    g = v > c3
    nc3 = jnp.where(g, v, c3)
    nj3 = jnp.where(g, idx, j3)
    v, idx = jnp.where(g, c3, v), jnp.where(g, j3, idx)
    g = v > c4
    nc4 = jnp.where(g, v, c4)
    nj4 = jnp.where(g, idx, j4)
    return nc1, nc2, nc3, nc4, nj1, nj2, nj3, nj4


def _sc_body(fp_hbm, bb_hbm, pair_hbm, fp_v, bb_v, pair_v):
    b = lax.axis_index("s") * 2 + lax.axis_index("c")
    pltpu.sync_copy(fp_hbm.at[pl.ds(b * _FPW, _FPW)], fp_v)
    pltpu.sync_copy(bb_hbm.at[pl.ds(b * _BBW, _BBW)], bb_v)

    neg = jnp.full((_L,), _NEG, jnp.float32)
    zero = jnp.zeros((_L,), jnp.float32)
    zi = jnp.zeros((_L,), jnp.int32)
    lane = lax.iota(jnp.int32, _L)
    lane_off0 = (lane >> 3) * (_NQT * 1024) + (lane & 7) * 128

    for g in range(_NG):
        lane_off = lane_off0 + (2 * g) * (_NQT * 1024)

        def chunk_body(j, carry, lane_off=lane_off):
            c1, c2, c3, c4, j1, j2, j3, j4 = carry
            base = lane_off + (j >> 3) * 1024 + (j & 7) * _CH
            m = neg
            for t in range(_CH):
                m = jnp.maximum(m, plsc.load_gather(fp_v, [base + t]))
            return _insert4(m, zi + j, c1, c2, c3, c4, j1, j2, j3, j4)

        carry = (neg, neg, neg, neg, zi, zi, zi, zi)
        c1, c2, c3, c4, j1, j2, j3, j4 = lax.fori_loop(
            0, _NCH, chunk_body, carry)

        sa, sb, sc, sd = j1, j2, j3, j4
        sa, sb = jnp.minimum(sa, sb), jnp.maximum(sa, sb)
        sc, sd = jnp.minimum(sc, sd), jnp.maximum(sc, sd)
        sa, sc = jnp.minimum(sa, sc), jnp.maximum(sa, sc)
        sb, sd = jnp.minimum(sb, sd), jnp.maximum(sb, sd)
        sb, sc = jnp.minimum(sb, sc), jnp.maximum(sb, sc)

        carry2 = (neg, neg, neg, neg, zi, zi, zi, zi)
        for jk in (sa, sb, sc, sd):
            rowbase = jk * _CH
            addrbase = lane_off + (jk >> 3) * 1024 + (jk & 7) * _CH

            def resc(t, carry, rowbase=rowbase, addrbase=addrbase):
                m1, m2, m3, m4, i1, i2, i3, i4 = carry
                v = plsc.load_gather(fp_v, [addrbase + t])
                return _insert4(v, rowbase + t,
                                m1, m2, m3, m4, i1, i2, i3, i4)

            carry2 = lax.fori_loop(0, _CH, resc, carry2)
        m1, m2, m3, m4, i1, i2, i3, i4 = carry2

        ba = [(ik >> 7) * 1024 + (ik & 127) for ik in (i1, i2, i3, i4)]
        g0 = [plsc.load_gather(bb_v, [ba[0] + d * 128]) for d in range(4)]
        s = zero
        for k in (1, 2, 3):
            for d in range(4):
                s = s + jnp.abs(
                    plsc.load_gather(bb_v, [ba[k] + d * 128]) - g0[d])
        pair_v[pl.ds(g * _L, _L)] = s * 0.25

    pltpu.sync_copy(pair_v, pair_hbm.at[pl.ds(b * _C, _C)])


_sc_topk_cache = []


def _get_sc_topk():
    if not _sc_topk_cache:
        mesh = plsc.VectorSubcoreMesh(
            core_axis_name="c", subcore_axis_name="s",
            num_cores=2, num_subcores=16)
        _sc_topk_cache.append(pl.kernel(
            _sc_body,
            out_type=jax.ShapeDtypeStruct((_B * _C,), jnp.float32),
            mesh=mesh,
            scratch_types=[
                pltpu.VMEM((_FPW,), jnp.float32),
                pltpu.VMEM((_BBW,), jnp.float32),
                pltpu.VMEM((_C,), jnp.float32),
            ],
            compiler_params=pltpu.CompilerParams(
                needs_layout_passes=False,
                use_tc_tiling_on_sc=False,
            ),
        ))
    return _sc_topk_cache[0]


def _finish_body(sums_ref, pair_ref, lab_ref, a1_ref, a2_ref, warm_ref,
                 tot_ref, mil_ref, areg_ref, box_ref):
    s = sums_ref[...]
    labv = lab_ref[...]
    preds = jnp.clip(s, 0.0, 1.0)
    log_p = jnp.maximum(jnp.log(preds), -100.0)
    log_1mp = jnp.maximum(jnp.log(1.0 - preds), -100.0)
    mil = -jnp.mean(labv * log_p + (1.0 - labv) * log_1mp)
    a1 = a1_ref[...]
    a2 = a2_ref[...]
    areg = 0.01 * 0.5 * (jnp.mean((a1 - 0.5) ** 2)
                         + jnp.mean((a2 - 0.5) ** 2))
    warm = warm_ref[0, 0]
    pairsum = jnp.sum(pair_ref[...] * labv)
    valid = jnp.sum(labv) * 3.0
    box = warm * (pairsum / jnp.maximum(valid, 1.0))
    tot_ref[0, 0] = mil + areg + box
    mil_ref[0, 0] = mil
    areg_ref[0, 0] = areg
    box_ref[0, 0] = box


def kernel(final_prob, bboxes, alpha_1, alpha_2, image_labels,
           current_epoch, warmup_epochs):
    fpt = jnp.transpose(final_prob, (0, 2, 1))
    bbt = jnp.transpose(bboxes, (0, 2, 1))
    fp5, bb4, sums = pl.pallas_call(
        _prep_body,
        grid=(_B,),
        in_specs=[
            pl.BlockSpec((1, _C, _Q), lambda b: (b, 0, 0)),
            pl.BlockSpec((1, 4, _Q), lambda b: (b, 0, 0)),
        ],
        out_specs=[
            pl.BlockSpec((1, _NCT, _NQT, 8, 128), lambda b: (b, 0, 0, 0, 0)),
            pl.BlockSpec((1, _NQT, 8, 128), lambda b: (b, 0, 0, 0)),
            pl.BlockSpec((1, 1, _C), lambda b: (b, 0, 0)),
        ],
        out_shape=[
            jax.ShapeDtypeStruct((_B, _NCT, _NQT, 8, 128), jnp.float32),
            jax.ShapeDtypeStruct((_B, _NQT, 8, 128), jnp.float32),
            jax.ShapeDtypeStruct((_B, 1, _C), jnp.float32),
        ],
    )(fpt, bbt)
    sums = sums.reshape(_B, _C)
    pair = _get_sc_topk()(fp5.reshape(_B * _FPW), bb4.reshape(_B * _BBW))
    pair = pair.reshape(_B, _C)
    labv = image_labels.astype(jnp.float32)
    a1 = alpha_1.reshape(1, _B)
    a2 = alpha_2.reshape(1, _B)
    warm = (jnp.asarray(current_epoch, jnp.int32)
            >= jnp.asarray(warmup_epochs, jnp.int32))
    warm = warm.astype(jnp.float32).reshape(1, 1)
    tot, mil, areg, box = pl.pallas_call(
        _finish_body,
        out_shape=[jax.ShapeDtypeStruct((1, 1), jnp.float32)] * 4,
        out_specs=[pl.BlockSpec(memory_space=pltpu.SMEM)] * 4,
    )(sums, pair, labv, a1, a2, warm)
    return (tot[0, 0], mil[0, 0], areg[0, 0], box[0, 0])


# speedup vs baseline: 1.7655x; 1.3981x over previous
import jax
import jax.numpy as jnp
from jax import lax
from jax.experimental import pallas as pl
from jax.experimental.pallas import tpu as pltpu
from jax.experimental.pallas import tpu_sc as plsc

_B, _Q, _C = 32, 900, 80
_L = 16
_CH = 16
_NCH = 57
_QP = _NCH * _CH
_NT = _QP // 8
_NQT = 8
_FPW = _NT * 1024
_BBW = _NQT * 1024
_NG = _C // _L
_NEG = -3.0e38


def _prep_body(x_ref, y_ref, fp4_ref, bb4_ref, sums_ref):
    x = x_ref[0]
    sums_ref[0, 0] = jnp.sum(x, axis=1)
    xt = jnp.transpose(x)
    xp = jnp.concatenate(
        [xt, jnp.full((_QP - _Q, _C), _NEG, jnp.float32)], axis=0)
    fp4_ref[0, :, :, 0:80] = xp.reshape(_NT, 8, _C)
    y = y_ref[0]
    for qt in range(7):
        bb4_ref[0, qt, 0:4, :] = y[:, 128 * qt:128 * qt + 128]
    bb4_ref[0, 7, 0:4, 0:4] = y[:, 896:_Q]


def _insert4(v, idx, c1, c2, c3, c4, j1, j2, j3, j4):
    g = v > c1
    nc1 = jnp.where(g, v, c1)
    nj1 = jnp.where(g, idx, j1)
    v, idx = jnp.where(g, c1, v), jnp.where(g, j1, idx)
    g = v > c2
    nc2 = jnp.where(g, v, c2)
    nj2 = jnp.where(g, idx, j2)
    v, idx = jnp.where(g, c2, v), jnp.where(g, j2, idx)
    g = v > c3
    nc3 = jnp.where(g, v, c3)
    nj3 = jnp.where(g, idx, j3)
    v, idx = jnp.where(g, c3, v), jnp.where(g, j3, idx)
    g = v > c4
    nc4 = jnp.where(g, v, c4)
    nj4 = jnp.where(g, idx, j4)
    return nc1, nc2, nc3, nc4, nj1, nj2, nj3, nj4


def _sc_body(fp_hbm, bb_hbm, pair_hbm, fp_v, bb_v, pair_v):
    b = lax.axis_index("s") * 2 + lax.axis_index("c")
    pltpu.sync_copy(fp_hbm.at[pl.ds(b * _FPW, _FPW)], fp_v)
    pltpu.sync_copy(bb_hbm.at[pl.ds(b * _BBW, _BBW)], bb_v)

    neg = jnp.full((_L,), _NEG, jnp.float32)
    zero = jnp.zeros((_L,), jnp.float32)
    zi = jnp.zeros((_L,), jnp.int32)
    lane = lax.iota(jnp.int32, _L)

    for g in range(_NG):
        col0 = g * _L

        def chunk_body(j, carry, col0=col0):
            c1, c2, c3, c4, j1, j2, j3, j4 = carry
            base = j * 2048 + col0
            m = neg
            for t in range(_CH):
                off = base + (t // 8) * 1024 + (t % 8) * 128
                m = jnp.maximum(m, fp_v[pl.ds(off, _L)])
            return _insert4(m, zi + j, c1, c2, c3, c4, j1, j2, j3, j4)

        carry = (neg, neg, neg, neg, zi, zi, zi, zi)
        c1, c2, c3, c4, j1, j2, j3, j4 = lax.fori_loop(
            0, _NCH, chunk_body, carry)

        sa, sb, sc, sd = j1, j2, j3, j4
        sa, sb = jnp.minimum(sa, sb), jnp.maximum(sa, sb)
        sc, sd = jnp.minimum(sc, sd), jnp.maximum(sc, sd)
        sa, sc = jnp.minimum(sa, sc), jnp.maximum(sa, sc)
        sb, sd = jnp.minimum(sb, sd), jnp.maximum(sb, sd)
        sb, sc = jnp.minimum(sb, sc), jnp.maximum(sb, sc)

        colv = lane + col0
        carry2 = (neg, neg, neg, neg, zi, zi, zi, zi)
        for jk in (sa, sb, sc, sd):
            rowbase = jk * _CH
            addrbase = jk * 2048 + colv

            def resc(t, carry, rowbase=rowbase, addrbase=addrbase):
                m1, m2, m3, m4, i1, i2, i3, i4 = carry
                off = (t // 8) * 1024 + (t % 8) * 128
                v = plsc.load_gather(fp_v, [addrbase + off])
                return _insert4(v, rowbase + t,
                                m1, m2, m3, m4, i1, i2, i3, i4)

            carry2 = lax.fori_loop(0, _CH, resc, carry2)
        m1, m2, m3, m4, i1, i2, i3, i4 = carry2

        ba = [(ik >> 7) * 1024 + (ik & 127) for ik in (i1, i2, i3, i4)]
        g0 = [plsc.load_gather(bb_v, [ba[0] + d * 128]) for d in range(4)]
        s = zero
        for k in (1, 2, 3):
            for d in range(4):
                s = s + jnp.abs(
                    plsc.load_gather(bb_v, [ba[k] + d * 128]) - g0[d])
        pair_v[pl.ds(col0, _L)] = s * 0.25

    pltpu.sync_copy(pair_v, pair_hbm.at[pl.ds(b * _C, _C)])


_sc_topk_cache = []


def _get_sc_topk():
    if not _sc_topk_cache:
        mesh = plsc.VectorSubcoreMesh(
            core_axis_name="c", subcore_axis_name="s",
            num_cores=2, num_subcores=16)
        _sc_topk_cache.append(pl.kernel(
            _sc_body,
            out_type=jax.ShapeDtypeStruct((_B * _C,), jnp.float32),
            mesh=mesh,
            scratch_types=[
                pltpu.VMEM((_FPW,), jnp.float32),
                pltpu.VMEM((_BBW,), jnp.float32),
                pltpu.VMEM((_C,), jnp.float32),
            ],
            compiler_params=pltpu.CompilerParams(
                needs_layout_passes=False,
                use_tc_tiling_on_sc=False,
            ),
        ))
    return _sc_topk_cache[0]


def _finish_body(sums_ref, pair_ref, lab_ref, a1_ref, a2_ref, warm_ref,
                 tot_ref, mil_ref, areg_ref, box_ref):
    s = sums_ref[...]
    labv = lab_ref[...]
    preds = jnp.clip(s, 0.0, 1.0)
    log_p = jnp.maximum(jnp.log(preds), -100.0)
    log_1mp = jnp.maximum(jnp.log(1.0 - preds), -100.0)
    mil = -jnp.mean(labv * log_p + (1.0 - labv) * log_1mp)
    a1 = a1_ref[...]
    a2 = a2_ref[...]
    areg = 0.01 * 0.5 * (jnp.mean((a1 - 0.5) ** 2)
                         + jnp.mean((a2 - 0.5) ** 2))
    warm = warm_ref[0, 0]
    pairsum = jnp.sum(pair_ref[...] * labv)
    valid = jnp.sum(labv) * 3.0
    box = warm * (pairsum / jnp.maximum(valid, 1.0))
    tot_ref[0, 0] = mil + areg + box
    mil_ref[0, 0] = mil
    areg_ref[0, 0] = areg
    box_ref[0, 0] = box


def kernel(final_prob, bboxes, alpha_1, alpha_2, image_labels,
           current_epoch, warmup_epochs):
    fpt = jnp.transpose(final_prob, (0, 2, 1))
    bbt = jnp.transpose(bboxes, (0, 2, 1))
    fp4, bb4, sums = pl.pallas_call(
        _prep_body,
        grid=(_B,),
        in_specs=[
            pl.BlockSpec((1, _C, _Q), lambda b: (b, 0, 0)),
            pl.BlockSpec((1, 4, _Q), lambda b: (b, 0, 0)),
        ],
        out_specs=[
            pl.BlockSpec((1, _NT, 8, 128), lambda b: (b, 0, 0, 0)),
            pl.BlockSpec((1, _NQT, 8, 128), lambda b: (b, 0, 0, 0)),
            pl.BlockSpec((1, 1, _C), lambda b: (b, 0, 0)),
        ],
        out_shape=[
            jax.ShapeDtypeStruct((_B, _NT, 8, 128), jnp.float32),
            jax.ShapeDtypeStruct((_B, _NQT, 8, 128), jnp.float32),
            jax.ShapeDtypeStruct((_B, 1, _C), jnp.float32),
        ],
    )(fpt, bbt)
    sums = sums.reshape(_B, _C)
    pair = _get_sc_topk()(fp4.reshape(_B * _FPW), bb4.reshape(_B * _BBW))
    pair = pair.reshape(_B, _C)
    labv = image_labels.astype(jnp.float32)
    a1 = alpha_1.reshape(1, _B)
    a2 = alpha_2.reshape(1, _B)
    warm = (jnp.asarray(current_epoch, jnp.int32)
            >= jnp.asarray(warmup_epochs, jnp.int32))
    warm = warm.astype(jnp.float32).reshape(1, 1)
    tot, mil, areg, box = pl.pallas_call(
        _finish_body,
        out_shape=[jax.ShapeDtypeStruct((1, 1), jnp.float32)] * 4,
        out_specs=[pl.BlockSpec(memory_space=pltpu.SMEM)] * 4,
    )(sums, pair, labv, a1, a2, warm)
    return (tot[0, 0], mil[0, 0], areg[0, 0], box[0, 0])


# speedup vs baseline: 2.2317x; 1.2640x over previous
import jax
import jax.numpy as jnp
from jax import lax
from jax.experimental import pallas as pl
from jax.experimental.pallas import tpu as pltpu
from jax.experimental.pallas import tpu_sc as plsc

_B, _Q, _C = 32, 900, 80
_L = 16
_CH = 16
_NCH = 57
_QP = _NCH * _CH
_NT = _QP // 8
_NQT = 8
_FPW = _NT * 1024
_BBW = _NQT * 1024
_NG = _C // _L
_NEG = -3.0e38


_PB = 4


def _prep_body(x_ref, y_ref, fp4_ref, bb4_ref, sums_ref):
    for i in range(_PB):
        x = x_ref[i]
        sums_ref[i, 0] = jnp.sum(x, axis=1)
        xt = jnp.transpose(x)
        xp = jnp.concatenate(
            [xt, jnp.full((_QP - _Q, _C), _NEG, jnp.float32)], axis=0)
        fp4_ref[i, :, :, 0:80] = xp.reshape(_NT, 8, _C)
        y = y_ref[i]
        for qt in range(7):
            bb4_ref[i, qt, 0:4, :] = y[:, 128 * qt:128 * qt + 128]
        bb4_ref[i, 7, 0:4, 0:4] = y[:, 896:_Q]


def _insert4(v, idx, c1, c2, c3, c4, j1, j2, j3, j4):
    g = v > c1
    nc1 = jnp.where(g, v, c1)
    nj1 = jnp.where(g, idx, j1)
    v, idx = jnp.where(g, c1, v), jnp.where(g, j1, idx)
    g = v > c2
    nc2 = jnp.where(g, v, c2)
    nj2 = jnp.where(g, idx, j2)
    v, idx = jnp.where(g, c2, v), jnp.where(g, j2, idx)
    g = v > c3
    nc3 = jnp.where(g, v, c3)
    nj3 = jnp.where(g, idx, j3)
    v, idx = jnp.where(g, c3, v), jnp.where(g, j3, idx)
    g = v > c4
    nc4 = jnp.where(g, v, c4)
    nj4 = jnp.where(g, idx, j4)
    return nc1, nc2, nc3, nc4, nj1, nj2, nj3, nj4


def _sc_body(fp_hbm, bb_hbm, pair_hbm, fp_v, bb_v, pair_v):
    b = lax.axis_index("s") * 2 + lax.axis_index("c")
    pltpu.sync_copy(fp_hbm.at[pl.ds(b * _FPW, _FPW)], fp_v)
    pltpu.sync_copy(bb_hbm.at[pl.ds(b * _BBW, _BBW)], bb_v)

    neg = jnp.full((_L,), _NEG, jnp.float32)
    zero = jnp.zeros((_L,), jnp.float32)
    zi = jnp.zeros((_L,), jnp.int32)
    lane = lax.iota(jnp.int32, _L)

    for g in range(_NG):
        col0 = g * _L

        def chunk_body(j, carry, col0=col0):
            c1, c2, c3, c4, j1, j2, j3, j4 = carry
            base = j * 2048 + col0
            m = neg
            for t in range(_CH):
                off = base + (t // 8) * 1024 + (t % 8) * 128
                m = jnp.maximum(m, fp_v[pl.ds(off, _L)])
            return _insert4(m, zi + j, c1, c2, c3, c4, j1, j2, j3, j4)

        carry = (neg, neg, neg, neg, zi, zi, zi, zi)
        c1, c2, c3, c4, j1, j2, j3, j4 = lax.fori_loop(
            0, _NCH, chunk_body, carry)

        sa, sb, sc, sd = j1, j2, j3, j4
        sa, sb = jnp.minimum(sa, sb), jnp.maximum(sa, sb)
        sc, sd = jnp.minimum(sc, sd), jnp.maximum(sc, sd)
        sa, sc = jnp.minimum(sa, sc), jnp.maximum(sa, sc)
        sb, sd = jnp.minimum(sb, sd), jnp.maximum(sb, sd)
        sb, sc = jnp.minimum(sb, sc), jnp.maximum(sb, sc)

        colv = lane + col0
        carry2 = (neg, neg, neg, neg, zi, zi, zi, zi)
        for jk in (sa, sb, sc, sd):
            rowbase = jk * _CH
            addrbase = jk * 2048 + colv

            def resc(t, carry, rowbase=rowbase, addrbase=addrbase):
                m1, m2, m3, m4, i1, i2, i3, i4 = carry
                off = (t // 8) * 1024 + (t % 8) * 128
                v = plsc.load_gather(fp_v, [addrbase + off])
                return _insert4(v, rowbase + t,
                                m1, m2, m3, m4, i1, i2, i3, i4)

            carry2 = lax.fori_loop(0, _CH, resc, carry2)
        m1, m2, m3, m4, i1, i2, i3, i4 = carry2

        ba = [(ik >> 7) * 1024 + (ik & 127) for ik in (i1, i2, i3, i4)]
        g0 = [plsc.load_gather(bb_v, [ba[0] + d * 128]) for d in range(4)]
        s = zero
        for k in (1, 2, 3):
            for d in range(4):
                s = s + jnp.abs(
                    plsc.load_gather(bb_v, [ba[k] + d * 128]) - g0[d])
        pair_v[pl.ds(col0, _L)] = s * 0.25

    pltpu.sync_copy(pair_v, pair_hbm.at[pl.ds(b * _C, _C)])


_sc_topk_cache = []


def _get_sc_topk():
    if not _sc_topk_cache:
        mesh = plsc.VectorSubcoreMesh(
            core_axis_name="c", subcore_axis_name="s",
            num_cores=2, num_subcores=16)
        _sc_topk_cache.append(pl.kernel(
            _sc_body,
            out_type=jax.ShapeDtypeStruct((_B * _C,), jnp.float32),
            mesh=mesh,
            scratch_types=[
                pltpu.VMEM((_FPW,), jnp.float32),
                pltpu.VMEM((_BBW,), jnp.float32),
                pltpu.VMEM((_C,), jnp.float32),
            ],
            compiler_params=pltpu.CompilerParams(
                needs_layout_passes=False,
                use_tc_tiling_on_sc=False,
            ),
        ))
    return _sc_topk_cache[0]


def _finish_body(sums_ref, pair_ref, lab_ref, a1_ref, a2_ref, warm_ref,
                 tot_ref, mil_ref, areg_ref, box_ref):
    s = sums_ref[...]
    labv = lab_ref[...]
    preds = jnp.clip(s, 0.0, 1.0)
    log_p = jnp.maximum(jnp.log(preds), -100.0)
    log_1mp = jnp.maximum(jnp.log(1.0 - preds), -100.0)
    mil = -jnp.mean(labv * log_p + (1.0 - labv) * log_1mp)
    a1 = a1_ref[...]
    a2 = a2_ref[...]
    areg = 0.01 * 0.5 * (jnp.mean((a1 - 0.5) ** 2)
                         + jnp.mean((a2 - 0.5) ** 2))
    warm = warm_ref[0, 0]
    pairsum = jnp.sum(pair_ref[...] * labv)
    valid = jnp.sum(labv) * 3.0
    box = warm * (pairsum / jnp.maximum(valid, 1.0))
    tot_ref[0, 0] = mil + areg + box
    mil_ref[0, 0] = mil
    areg_ref[0, 0] = areg
    box_ref[0, 0] = box


def kernel(final_prob, bboxes, alpha_1, alpha_2, image_labels,
           current_epoch, warmup_epochs):
    fpt = jnp.transpose(final_prob, (0, 2, 1))
    bbt = jnp.transpose(bboxes, (0, 2, 1))
    fp4, bb4, sums = pl.pallas_call(
        _prep_body,
        grid=(_B // _PB,),
        in_specs=[
            pl.BlockSpec((_PB, _C, _Q), lambda b: (b, 0, 0)),
            pl.BlockSpec((_PB, 4, _Q), lambda b: (b, 0, 0)),
        ],
        out_specs=[
            pl.BlockSpec((_PB, _NT, 8, 128), lambda b: (b, 0, 0, 0)),
            pl.BlockSpec((_PB, _NQT, 8, 128), lambda b: (b, 0, 0, 0)),
            pl.BlockSpec((_PB, 1, _C), lambda b: (b, 0, 0)),
        ],
        out_shape=[
            jax.ShapeDtypeStruct((_B, _NT, 8, 128), jnp.float32),
            jax.ShapeDtypeStruct((_B, _NQT, 8, 128), jnp.float32),
            jax.ShapeDtypeStruct((_B, 1, _C), jnp.float32),
        ],
    )(fpt, bbt)
    sums = sums.reshape(_B, _C)
    pair = _get_sc_topk()(fp4.reshape(_B * _FPW), bb4.reshape(_B * _BBW))
    pair = pair.reshape(_B, _C)
    labv = image_labels.astype(jnp.float32)
    a1 = alpha_1.reshape(1, _B)
    a2 = alpha_2.reshape(1, _B)
    warm = (jnp.asarray(current_epoch, jnp.int32)
            >= jnp.asarray(warmup_epochs, jnp.int32))
    warm = warm.astype(jnp.float32).reshape(1, 1)
    tot, mil, areg, box = pl.pallas_call(
        _finish_body,
        out_shape=[jax.ShapeDtypeStruct((1, 1), jnp.float32)] * 4,
        out_specs=[pl.BlockSpec(memory_space=pltpu.SMEM)] * 4,
    )(sums, pair, labv, a1, a2, warm)
    return (tot[0, 0], mil[0, 0], areg[0, 0], box[0, 0])


# speedup vs baseline: 2.3257x; 1.0421x over previous
import jax
import jax.numpy as jnp
from jax import lax
from jax.experimental import pallas as pl
from jax.experimental.pallas import tpu as pltpu
from jax.experimental.pallas import tpu_sc as plsc

_B, _Q, _C = 32, 900, 80
_L = 16
_CH = 16
_NCH = 57
_QP = _NCH * _CH
_NT = _QP // 8
_NQT = 8
_FPW = _NT * 1024
_BBW = _NQT * 1024
_NG = _C // _L
_NEG = -3.0e38


_PB = 8


def _prep_body(x_ref, y_ref, fp4_ref, bb4_ref, sums_ref):
    for i in range(_PB):
        x = x_ref[i]
        sums_ref[i, 0] = jnp.sum(x, axis=1)
        xt = jnp.transpose(x)
        xp = jnp.concatenate(
            [xt, jnp.full((_QP - _Q, _C), _NEG, jnp.float32)], axis=0)
        fp4_ref[i, :, :, 0:80] = xp.reshape(_NT, 8, _C)
        y = y_ref[i]
        for qt in range(7):
            bb4_ref[i, qt, 0:4, :] = y[:, 128 * qt:128 * qt + 128]
        bb4_ref[i, 7, 0:4, 0:4] = y[:, 896:_Q]


def _insert4(v, idx, c1, c2, c3, c4, j1, j2, j3, j4):
    g = v > c1
    nc1 = jnp.where(g, v, c1)
    nj1 = jnp.where(g, idx, j1)
    v, idx = jnp.where(g, c1, v), jnp.where(g, j1, idx)
    g = v > c2
    nc2 = jnp.where(g, v, c2)
    nj2 = jnp.where(g, idx, j2)
    v, idx = jnp.where(g, c2, v), jnp.where(g, j2, idx)
    g = v > c3
    nc3 = jnp.where(g, v, c3)
    nj3 = jnp.where(g, idx, j3)
    v, idx = jnp.where(g, c3, v), jnp.where(g, j3, idx)
    g = v > c4
    nc4 = jnp.where(g, v, c4)
    nj4 = jnp.where(g, idx, j4)
    return nc1, nc2, nc3, nc4, nj1, nj2, nj3, nj4


def _sc_body(fp_hbm, bb_hbm, pair_hbm, fp_v, bb_v, pair_v):
    b = lax.axis_index("s") * 2 + lax.axis_index("c")
    pltpu.sync_copy(fp_hbm.at[pl.ds(b * _FPW, _FPW)], fp_v)
    pltpu.sync_copy(bb_hbm.at[pl.ds(b * _BBW, _BBW)], bb_v)

    neg = jnp.full((_L,), _NEG, jnp.float32)
    zero = jnp.zeros((_L,), jnp.float32)
    zi = jnp.zeros((_L,), jnp.int32)
    lane = lax.iota(jnp.int32, _L)

    for g in range(_NG):
        col0 = g * _L

        def chunk_body(j, carry, col0=col0):
            c1, c2, c3, c4, j1, j2, j3, j4 = carry
            base = j * 2048 + col0
            m = neg
            for t in range(_CH):
                off = base + (t // 8) * 1024 + (t % 8) * 128
                m = jnp.maximum(m, fp_v[pl.ds(off, _L)])
            return _insert4(m, zi + j, c1, c2, c3, c4, j1, j2, j3, j4)

        carry = (neg, neg, neg, neg, zi, zi, zi, zi)
        c1, c2, c3, c4, j1, j2, j3, j4 = lax.fori_loop(
            0, _NCH, chunk_body, carry)

        sa, sb, sc, sd = j1, j2, j3, j4
        sa, sb = jnp.minimum(sa, sb), jnp.maximum(sa, sb)
        sc, sd = jnp.minimum(sc, sd), jnp.maximum(sc, sd)
        sa, sc = jnp.minimum(sa, sc), jnp.maximum(sa, sc)
        sb, sd = jnp.minimum(sb, sd), jnp.maximum(sb, sd)
        sb, sc = jnp.minimum(sb, sc), jnp.maximum(sb, sc)

        colv = lane + col0
        carry2 = (neg, neg, neg, neg, zi, zi, zi, zi)
        for jk in (sa, sb, sc, sd):
            rowbase = jk * _CH
            addrbase = jk * 2048 + colv

            def resc(t, carry, rowbase=rowbase, addrbase=addrbase):
                m1, m2, m3, m4, i1, i2, i3, i4 = carry
                off = (t // 8) * 1024 + (t % 8) * 128
                v = plsc.load_gather(fp_v, [addrbase + off])
                return _insert4(v, rowbase + t,
                                m1, m2, m3, m4, i1, i2, i3, i4)

            carry2 = lax.fori_loop(0, _CH, resc, carry2)
        m1, m2, m3, m4, i1, i2, i3, i4 = carry2

        ba = [(ik >> 7) * 1024 + (ik & 127) for ik in (i1, i2, i3, i4)]
        g0 = [plsc.load_gather(bb_v, [ba[0] + d * 128]) for d in range(4)]
        s = zero
        for k in (1, 2, 3):
            for d in range(4):
                s = s + jnp.abs(
                    plsc.load_gather(bb_v, [ba[k] + d * 128]) - g0[d])
        pair_v[pl.ds(col0, _L)] = s * 0.25

    pltpu.sync_copy(pair_v, pair_hbm.at[pl.ds(b * _C, _C)])


_sc_topk_cache = []


def _get_sc_topk():
    if not _sc_topk_cache:
        mesh = plsc.VectorSubcoreMesh(
            core_axis_name="c", subcore_axis_name="s",
            num_cores=2, num_subcores=16)
        _sc_topk_cache.append(pl.kernel(
            _sc_body,
            out_type=jax.ShapeDtypeStruct((_B * _C,), jnp.float32),
            mesh=mesh,
            scratch_types=[
                pltpu.VMEM((_FPW,), jnp.float32),
                pltpu.VMEM((_BBW,), jnp.float32),
                pltpu.VMEM((_C,), jnp.float32),
            ],
            compiler_params=pltpu.CompilerParams(
                needs_layout_passes=False,
                use_tc_tiling_on_sc=False,
            ),
        ))
    return _sc_topk_cache[0]


def _finish_body(sums_ref, pair_ref, lab_ref, a1_ref, a2_ref, warm_ref,
                 tot_ref, mil_ref, areg_ref, box_ref):
    s = sums_ref[...]
    labv = lab_ref[...]
    preds = jnp.clip(s, 0.0, 1.0)
    log_p = jnp.maximum(jnp.log(preds), -100.0)
    log_1mp = jnp.maximum(jnp.log(1.0 - preds), -100.0)
    mil = -jnp.mean(labv * log_p + (1.0 - labv) * log_1mp)
    a1 = a1_ref[...]
    a2 = a2_ref[...]
    areg = 0.01 * 0.5 * (jnp.mean((a1 - 0.5) ** 2)
                         + jnp.mean((a2 - 0.5) ** 2))
    warm = warm_ref[0, 0]
    pairsum = jnp.sum(pair_ref[...] * labv)
    valid = jnp.sum(labv) * 3.0
    box = warm * (pairsum / jnp.maximum(valid, 1.0))
    tot_ref[0, 0] = mil + areg + box
    mil_ref[0, 0] = mil
    areg_ref[0, 0] = areg
    box_ref[0, 0] = box


def kernel(final_prob, bboxes, alpha_1, alpha_2, image_labels,
           current_epoch, warmup_epochs):
    fpt = jnp.transpose(final_prob, (0, 2, 1))
    bbt = jnp.transpose(bboxes, (0, 2, 1))
    fp4, bb4, sums = pl.pallas_call(
        _prep_body,
        grid=(_B // _PB,),
        in_specs=[
            pl.BlockSpec((_PB, _C, _Q), lambda b: (b, 0, 0)),
            pl.BlockSpec((_PB, 4, _Q), lambda b: (b, 0, 0)),
        ],
        out_specs=[
            pl.BlockSpec((_PB, _NT, 8, 128), lambda b: (b, 0, 0, 0)),
            pl.BlockSpec((_PB, _NQT, 8, 128), lambda b: (b, 0, 0, 0)),
            pl.BlockSpec((_PB, 1, _C), lambda b: (b, 0, 0)),
        ],
        out_shape=[
            jax.ShapeDtypeStruct((_B, _NT, 8, 128), jnp.float32),
            jax.ShapeDtypeStruct((_B, _NQT, 8, 128), jnp.float32),
            jax.ShapeDtypeStruct((_B, 1, _C), jnp.float32),
        ],
    )(fpt, bbt)
    sums = sums.reshape(_B, _C)
    pair = _get_sc_topk()(fp4.reshape(_B * _FPW), bb4.reshape(_B * _BBW))
    pair = pair.reshape(_B, _C)
    labv = image_labels.astype(jnp.float32)
    a1 = alpha_1.reshape(1, _B)
    a2 = alpha_2.reshape(1, _B)
    warm = (jnp.asarray(current_epoch, jnp.int32)
            >= jnp.asarray(warmup_epochs, jnp.int32))
    warm = warm.astype(jnp.float32).reshape(1, 1)
    tot, mil, areg, box = pl.pallas_call(
        _finish_body,
        out_shape=[jax.ShapeDtypeStruct((1, 1), jnp.float32)] * 4,
        out_specs=[pl.BlockSpec(memory_space=pltpu.SMEM)] * 4,
    )(sums, pair, labv, a1, a2, warm)
    return (tot[0, 0], mil[0, 0], areg[0, 0], box[0, 0])


# speedup vs baseline: 2.3747x; 1.0211x over previous
import jax
import jax.numpy as jnp
from jax import lax
from jax.experimental import pallas as pl
from jax.experimental.pallas import tpu as pltpu
from jax.experimental.pallas import tpu_sc as plsc

_B, _Q, _C = 32, 900, 80
_L = 16
_CH = 16
_NCH = 57
_QP = _NCH * _CH
_NT = _QP // 8
_NQT = 8
_FPW = _NT * 1024
_BBW = _NQT * 1024
_NG = _C // _L
_NEG = -3.0e38


_PB = 8


def _prep_body(x_ref, y_ref, fp4_ref, bb4_ref, sums_ref):
    for i in range(_PB):
        x = x_ref[i]
        sums_ref[i, 0] = jnp.sum(x, axis=1)
        xt = jnp.transpose(x)
        xp = jnp.concatenate(
            [xt, jnp.full((_QP - _Q, _C), _NEG, jnp.float32)], axis=0)
        fp4_ref[i, :, :, 0:80] = xp.reshape(_NT, 8, _C)
        y = y_ref[i]
        for qt in range(7):
            bb4_ref[i, qt, 0:4, :] = y[:, 128 * qt:128 * qt + 128]
        bb4_ref[i, 7, 0:4, 0:4] = y[:, 896:_Q]


def _insert4(v, idx, c1, c2, c3, c4, j1, j2, j3, j4):
    g = v > c1
    nc1 = jnp.where(g, v, c1)
    nj1 = jnp.where(g, idx, j1)
    v, idx = jnp.where(g, c1, v), jnp.where(g, j1, idx)
    g = v > c2
    nc2 = jnp.where(g, v, c2)
    nj2 = jnp.where(g, idx, j2)
    v, idx = jnp.where(g, c2, v), jnp.where(g, j2, idx)
    g = v > c3
    nc3 = jnp.where(g, v, c3)
    nj3 = jnp.where(g, idx, j3)
    v, idx = jnp.where(g, c3, v), jnp.where(g, j3, idx)
    g = v > c4
    nc4 = jnp.where(g, v, c4)
    nj4 = jnp.where(g, idx, j4)
    return nc1, nc2, nc3, nc4, nj1, nj2, nj3, nj4


def _sc_body(fp_hbm, bb_hbm, pair_hbm, fp_v, bb_v, pair_v, bb_sem):
    b = lax.axis_index("s") * 2 + lax.axis_index("c")
    bb_cp = pltpu.make_async_copy(
        bb_hbm.at[pl.ds(b * _BBW, _BBW)], bb_v, bb_sem)
    bb_cp.start()
    pltpu.sync_copy(fp_hbm.at[pl.ds(b * _FPW, _FPW)], fp_v)

    neg = jnp.full((_L,), _NEG, jnp.float32)
    zero = jnp.zeros((_L,), jnp.float32)
    zi = jnp.zeros((_L,), jnp.int32)
    lane = lax.iota(jnp.int32, _L)

    for g in range(_NG):
        col0 = g * _L

        def chunk_body(j, carry, col0=col0):
            c1, c2, c3, c4, j1, j2, j3, j4 = carry
            base = j * 2048 + col0
            m = neg
            for t in range(_CH):
                off = base + (t // 8) * 1024 + (t % 8) * 128
                m = jnp.maximum(m, fp_v[pl.ds(off, _L)])
            return _insert4(m, zi + j, c1, c2, c3, c4, j1, j2, j3, j4)

        carry = (neg, neg, neg, neg, zi, zi, zi, zi)
        c1, c2, c3, c4, j1, j2, j3, j4 = lax.fori_loop(
            0, _NCH, chunk_body, carry)

        sa, sb, sc, sd = j1, j2, j3, j4
        sa, sb = jnp.minimum(sa, sb), jnp.maximum(sa, sb)
        sc, sd = jnp.minimum(sc, sd), jnp.maximum(sc, sd)
        sa, sc = jnp.minimum(sa, sc), jnp.maximum(sa, sc)
        sb, sd = jnp.minimum(sb, sd), jnp.maximum(sb, sd)
        sb, sc = jnp.minimum(sb, sc), jnp.maximum(sb, sc)

        if g == 0:
            bb_cp.wait()
        colv = lane + col0
        carry2 = (neg, neg, neg, neg, zi, zi, zi, zi)
        for jk in (sa, sb, sc, sd):
            rowbase = jk * _CH
            addrbase = jk * 2048 + colv

            def resc(t, carry, rowbase=rowbase, addrbase=addrbase):
                m1, m2, m3, m4, i1, i2, i3, i4 = carry
                off = (t // 8) * 1024 + (t % 8) * 128
                v = plsc.load_gather(fp_v, [addrbase + off])
                return _insert4(v, rowbase + t,
                                m1, m2, m3, m4, i1, i2, i3, i4)

            carry2 = lax.fori_loop(0, _CH, resc, carry2)
        m1, m2, m3, m4, i1, i2, i3, i4 = carry2

        ba = [(ik >> 7) * 1024 + (ik & 127) for ik in (i1, i2, i3, i4)]
        g0 = [plsc.load_gather(bb_v, [ba[0] + d * 128]) for d in range(4)]
        s = zero
        for k in (1, 2, 3):
            for d in range(4):
                s = s + jnp.abs(
                    plsc.load_gather(bb_v, [ba[k] + d * 128]) - g0[d])
        pair_v[pl.ds(col0, _L)] = s * 0.25

    pltpu.sync_copy(pair_v, pair_hbm.at[b])


_sc_topk_cache = []


def _get_sc_topk():
    if not _sc_topk_cache:
        mesh = plsc.VectorSubcoreMesh(
            core_axis_name="c", subcore_axis_name="s",
            num_cores=2, num_subcores=16)
        _sc_topk_cache.append(pl.kernel(
            _sc_body,
            out_type=jax.ShapeDtypeStruct((_B, _C), jnp.float32),
            mesh=mesh,
            scratch_types=[
                pltpu.VMEM((_FPW,), jnp.float32),
                pltpu.VMEM((_BBW,), jnp.float32),
                pltpu.VMEM((_C,), jnp.float32),
                pltpu.SemaphoreType.DMA,
            ],
            compiler_params=pltpu.CompilerParams(
                needs_layout_passes=False,
                use_tc_tiling_on_sc=False,
            ),
        ))
    return _sc_topk_cache[0]


def _finish_body(sums_ref, pair_ref, lab_ref, a1_ref, a2_ref, warm_ref,
                 tot_ref, mil_ref, areg_ref, box_ref):
    s = sums_ref[...]
    labv = lab_ref[...]
    preds = jnp.clip(s, 0.0, 1.0)
    log_p = jnp.maximum(jnp.log(preds), -100.0)
    log_1mp = jnp.maximum(jnp.log(1.0 - preds), -100.0)
    mil = -jnp.mean(labv * log_p + (1.0 - labv) * log_1mp)
    a1 = a1_ref[...]
    a2 = a2_ref[...]
    areg = 0.01 * 0.5 * (jnp.mean((a1 - 0.5) ** 2)
                         + jnp.mean((a2 - 0.5) ** 2))
    warm = warm_ref[0, 0]
    pairsum = jnp.sum(pair_ref[...] * labv)
    valid = jnp.sum(labv) * 3.0
    box = warm * (pairsum / jnp.maximum(valid, 1.0))
    tot_ref[0, 0] = mil + areg + box
    mil_ref[0, 0] = mil
    areg_ref[0, 0] = areg
    box_ref[0, 0] = box


def kernel(final_prob, bboxes, alpha_1, alpha_2, image_labels,
           current_epoch, warmup_epochs):
    fpt = jnp.transpose(final_prob, (0, 2, 1))
    bbt = jnp.transpose(bboxes, (0, 2, 1))
    fp4, bb4, sums = pl.pallas_call(
        _prep_body,
        grid=(_B // _PB,),
        in_specs=[
            pl.BlockSpec((_PB, _C, _Q), lambda b: (b, 0, 0)),
            pl.BlockSpec((_PB, 4, _Q), lambda b: (b, 0, 0)),
        ],
        out_specs=[
            pl.BlockSpec((_PB, _NT, 8, 128), lambda b: (b, 0, 0, 0)),
            pl.BlockSpec((_PB, _NQT, 8, 128), lambda b: (b, 0, 0, 0)),
            pl.BlockSpec((_PB, 1, _C), lambda b: (b, 0, 0)),
        ],
        out_shape=[
            jax.ShapeDtypeStruct((_B, _NT, 8, 128), jnp.float32),
            jax.ShapeDtypeStruct((_B, _NQT, 8, 128), jnp.float32),
            jax.ShapeDtypeStruct((_B, 1, _C), jnp.float32),
        ],
    )(fpt, bbt)
    sums = sums.reshape(_B, _C)
    pair = _get_sc_topk()(fp4.reshape(_B * _FPW), bb4.reshape(_B * _BBW))
    labv = image_labels.astype(jnp.float32)
    a1 = alpha_1.reshape(1, _B)
    a2 = alpha_2.reshape(1, _B)
    warm = (jnp.asarray(current_epoch, jnp.int32)
            >= jnp.asarray(warmup_epochs, jnp.int32))
    warm = warm.astype(jnp.float32).reshape(1, 1)
    tot, mil, areg, box = pl.pallas_call(
        _finish_body,
        out_shape=[jax.ShapeDtypeStruct((1, 1), jnp.float32)] * 4,
        out_specs=[pl.BlockSpec(memory_space=pltpu.SMEM)] * 4,
    )(sums, pair, labv, a1, a2, warm)
    return (tot[0, 0], mil[0, 0], areg[0, 0], box[0, 0])


# speedup vs baseline: 2.4194x; 1.0188x over previous
import jax
import jax.numpy as jnp
from jax import lax
from jax.experimental import pallas as pl
from jax.experimental.pallas import tpu as pltpu
from jax.experimental.pallas import tpu_sc as plsc

_B, _Q, _C = 32, 900, 80
_L = 16
_CH = 16
_NCH = 57
_QP = _NCH * _CH
_NT = _QP // 8
_NQT = 8
_FPW = _NT * 1024
_BBW = _NQT * 1024
_NG = _C // _L
_NEG = -3.0e38


_PB = 16


def _prep_body(x_ref, y_ref, fp4_ref, bb4_ref, sums_ref):
    for i in range(_PB):
        x = x_ref[i]
        sums_ref[i, 0] = jnp.sum(x, axis=1)
        xt = jnp.transpose(x)
        xp = jnp.concatenate(
            [xt, jnp.full((_QP - _Q, _C), _NEG, jnp.float32)], axis=0)
        fp4_ref[i, :, :, 0:80] = xp.reshape(_NT, 8, _C)
        y = y_ref[i]
        for qt in range(7):
            bb4_ref[i, qt, 0:4, :] = y[:, 128 * qt:128 * qt + 128]
        bb4_ref[i, 7, 0:4, 0:4] = y[:, 896:_Q]


def _insert4(v, idx, c1, c2, c3, c4, j1, j2, j3, j4):
    g = v > c1
    nc1 = jnp.where(g, v, c1)
    nj1 = jnp.where(g, idx, j1)
    v, idx = jnp.where(g, c1, v), jnp.where(g, j1, idx)
    g = v > c2
    nc2 = jnp.where(g, v, c2)
    nj2 = jnp.where(g, idx, j2)
    v, idx = jnp.where(g, c2, v), jnp.where(g, j2, idx)
    g = v > c3
    nc3 = jnp.where(g, v, c3)
    nj3 = jnp.where(g, idx, j3)
    v, idx = jnp.where(g, c3, v), jnp.where(g, j3, idx)
    g = v > c4
    nc4 = jnp.where(g, v, c4)
    nj4 = jnp.where(g, idx, j4)
    return nc1, nc2, nc3, nc4, nj1, nj2, nj3, nj4


def _sc_body(fp_hbm, bb_hbm, pair_hbm, fp_v, bb_v, pair_v, bb_sem):
    b = lax.axis_index("s") * 2 + lax.axis_index("c")
    bb_cp = pltpu.make_async_copy(
        bb_hbm.at[pl.ds(b * _BBW, _BBW)], bb_v, bb_sem)
    bb_cp.start()
    pltpu.sync_copy(fp_hbm.at[pl.ds(b * _FPW, _FPW)], fp_v)

    neg = jnp.full((_L,), _NEG, jnp.float32)
    zero = jnp.zeros((_L,), jnp.float32)
    zi = jnp.zeros((_L,), jnp.int32)
    lane = lax.iota(jnp.int32, _L)

    for g in range(_NG):
        col0 = g * _L

        def chunk_body(j, carry, col0=col0):
            c1, c2, c3, c4, j1, j2, j3, j4 = carry
            base = j * 2048 + col0
            m = neg
            for t in range(_CH):
                off = base + (t // 8) * 1024 + (t % 8) * 128
                m = jnp.maximum(m, fp_v[pl.ds(off, _L)])
            return _insert4(m, zi + j, c1, c2, c3, c4, j1, j2, j3, j4)

        carry = (neg, neg, neg, neg, zi, zi, zi, zi)
        c1, c2, c3, c4, j1, j2, j3, j4 = lax.fori_loop(
            0, _NCH, chunk_body, carry)

        sa, sb, sc, sd = j1, j2, j3, j4
        sa, sb = jnp.minimum(sa, sb), jnp.maximum(sa, sb)
        sc, sd = jnp.minimum(sc, sd), jnp.maximum(sc, sd)
        sa, sc = jnp.minimum(sa, sc), jnp.maximum(sa, sc)
        sb, sd = jnp.minimum(sb, sd), jnp.maximum(sb, sd)
        sb, sc = jnp.minimum(sb, sc), jnp.maximum(sb, sc)

        if g == 0:
            bb_cp.wait()
        colv = lane + col0
        carry2 = (neg, neg, neg, neg, zi, zi, zi, zi)
        for jk in (sa, sb, sc, sd):
            rowbase = jk * _CH
            addrbase = jk * 2048 + colv

            def resc(t, carry, rowbase=rowbase, addrbase=addrbase):
                m1, m2, m3, m4, i1, i2, i3, i4 = carry
                off = (t // 8) * 1024 + (t % 8) * 128
                v = plsc.load_gather(fp_v, [addrbase + off])
                return _insert4(v, rowbase + t,
                                m1, m2, m3, m4, i1, i2, i3, i4)

            carry2 = lax.fori_loop(0, _CH, resc, carry2)
        m1, m2, m3, m4, i1, i2, i3, i4 = carry2

        ba = [(ik >> 7) * 1024 + (ik & 127) for ik in (i1, i2, i3, i4)]
        g0 = [plsc.load_gather(bb_v, [ba[0] + d * 128]) for d in range(4)]
        s = zero
        for k in (1, 2, 3):
            for d in range(4):
                s = s + jnp.abs(
                    plsc.load_gather(bb_v, [ba[k] + d * 128]) - g0[d])
        pair_v[pl.ds(col0, _L)] = s * 0.25

    pltpu.sync_copy(pair_v, pair_hbm.at[b])


_sc_topk_cache = []


def _get_sc_topk():
    if not _sc_topk_cache:
        mesh = plsc.VectorSubcoreMesh(
            core_axis_name="c", subcore_axis_name="s",
            num_cores=2, num_subcores=16)
        _sc_topk_cache.append(pl.kernel(
            _sc_body,
            out_type=jax.ShapeDtypeStruct((_B, _C), jnp.float32),
            mesh=mesh,
            scratch_types=[
                pltpu.VMEM((_FPW,), jnp.float32),
                pltpu.VMEM((_BBW,), jnp.float32),
                pltpu.VMEM((_C,), jnp.float32),
                pltpu.SemaphoreType.DMA,
            ],
            compiler_params=pltpu.CompilerParams(
                needs_layout_passes=False,
                use_tc_tiling_on_sc=False,
            ),
        ))
    return _sc_topk_cache[0]


def _finish_body(sums_ref, pair_ref, lab_ref, a1_ref, a2_ref, warm_ref,
                 tot_ref, mil_ref, areg_ref, box_ref):
    s = sums_ref[...]
    labv = lab_ref[...]
    preds = jnp.clip(s, 0.0, 1.0)
    log_p = jnp.maximum(jnp.log(preds), -100.0)
    log_1mp = jnp.maximum(jnp.log(1.0 - preds), -100.0)
    mil = -jnp.mean(labv * log_p + (1.0 - labv) * log_1mp)
    a1 = a1_ref[...]
    a2 = a2_ref[...]
    areg = 0.01 * 0.5 * (jnp.mean((a1 - 0.5) ** 2)
                         + jnp.mean((a2 - 0.5) ** 2))
    warm = warm_ref[0, 0]
    pairsum = jnp.sum(pair_ref[...] * labv)
    valid = jnp.sum(labv) * 3.0
    box = warm * (pairsum / jnp.maximum(valid, 1.0))
    tot_ref[0, 0] = mil + areg + box
    mil_ref[0, 0] = mil
    areg_ref[0, 0] = areg
    box_ref[0, 0] = box


def kernel(final_prob, bboxes, alpha_1, alpha_2, image_labels,
           current_epoch, warmup_epochs):
    fpt = jnp.transpose(final_prob, (0, 2, 1))
    bbt = jnp.transpose(bboxes, (0, 2, 1))
    fp4, bb4, sums = pl.pallas_call(
        _prep_body,
        grid=(_B // _PB,),
        in_specs=[
            pl.BlockSpec((_PB, _C, _Q), lambda b: (b, 0, 0)),
            pl.BlockSpec((_PB, 4, _Q), lambda b: (b, 0, 0)),
        ],
        out_specs=[
            pl.BlockSpec((_PB, _NT, 8, 128), lambda b: (b, 0, 0, 0)),
            pl.BlockSpec((_PB, _NQT, 8, 128), lambda b: (b, 0, 0, 0)),
            pl.BlockSpec((_PB, 1, _C), lambda b: (b, 0, 0)),
        ],
        out_shape=[
            jax.ShapeDtypeStruct((_B, _NT, 8, 128), jnp.float32),
            jax.ShapeDtypeStruct((_B, _NQT, 8, 128), jnp.float32),
            jax.ShapeDtypeStruct((_B, 1, _C), jnp.float32),
        ],
    )(fpt, bbt)
    sums = sums.reshape(_B, _C)
    pair = _get_sc_topk()(fp4.reshape(_B * _FPW), bb4.reshape(_B * _BBW))
    labv = image_labels.astype(jnp.float32)
    a1 = alpha_1.reshape(1, _B)
    a2 = alpha_2.reshape(1, _B)
    warm = (jnp.asarray(current_epoch, jnp.int32)
            >= jnp.asarray(warmup_epochs, jnp.int32))
    warm = warm.astype(jnp.float32).reshape(1, 1)
    tot, mil, areg, box = pl.pallas_call(
        _finish_body,
        out_shape=[jax.ShapeDtypeStruct((1, 1), jnp.float32)] * 4,
        out_specs=[pl.BlockSpec(memory_space=pltpu.SMEM)] * 4,
    )(sums, pair, labv, a1, a2, warm)
    return (tot[0, 0], mil[0, 0], areg[0, 0], box[0, 0])
